# baseline (speedup 1.0000x reference)
import functools

import jax
import jax.numpy as jnp
from jax import lax
from jax.experimental import pallas as pl
from jax.experimental.pallas import tpu as pltpu
from jax.experimental.pallas import tpu_sc as plsc

B, N, DIN, DOUT, NKEEP = 8, 1024, 256, 256, 512
bf16, f32, i32 = jnp.bfloat16, jnp.float32, jnp.int32


def _tc_body(a_ref, x_ref, wr_ref, wm_ref, bm_ref,
             xo_ref, gid_ref, rnk_ref):
    b = pl.program_id(0)
    a = a_ref[0].astype(bf16)
    x = x_ref[0].astype(bf16)
    mt = lax.dot_general(x, a, (((0,), (1,)), ((), ())),
                         preferred_element_type=f32)
    m_bf = mt.T.astype(bf16)
    wr_bf = wr_ref[...].astype(bf16)
    t_col = jnp.dot(m_bf, wr_bf.T, preferred_element_type=f32)
    t_row = lax.dot_general(wr_bf.T, m_bf, (((0,), (1,)), ((), ())),
                            preferred_element_type=f32)

    wm_bf = wm_ref[...].astype(bf16)
    xo = jnp.dot(m_bf, wm_bf.T, preferred_element_type=f32) + bm_ref[...]
    xo_ref[0] = jnp.where(xo >= 0, xo, 0.01 * xo)

    ii = lax.broadcasted_iota(i32, (N, N), 0)
    jj = lax.broadcasted_iota(i32, (N, N), 1)
    beats_fe = (t_col > t_row) | ((t_col == t_row) & (ii < jj))
    rank_row = jnp.sum(jnp.where(beats_fe, 1.0, 0.0), axis=0,
                       keepdims=True)
    rnk_ref[0] = rank_row.astype(i32)
    beats_ef = (t_row > t_col) | ((t_row == t_col) & (jj < ii))
    rank_col = jnp.sum(jnp.where(beats_ef, 1.0, 0.0), axis=1,
                       keepdims=True)

    pmat = lax.broadcasted_iota(i32, (N, NKEEP), 1).astype(f32)
    emat = lax.broadcasted_iota(i32, (N, NKEEP), 0).astype(f32)
    onehot = rank_col == pmat
    idx_row = jnp.sum(jnp.where(onehot, emat, 0.0), axis=0,
                      keepdims=True)
    gid_ref[0] = idx_row.astype(i32) + b * N


def _tc_stage(A, X, Wr, Wm, bm):
    return pl.pallas_call(
        _tc_body,
        grid=(B,),
        in_specs=[
            pl.BlockSpec((1, N, N), lambda b: (b, 0, 0)),
            pl.BlockSpec((1, N, DIN), lambda b: (b, 0, 0)),
            pl.BlockSpec((1, DIN), lambda b: (0, 0)),
            pl.BlockSpec((DOUT, DIN), lambda b: (0, 0)),
            pl.BlockSpec((DOUT,), lambda b: (0,)),
        ],
        out_specs=[
            pl.BlockSpec((1, N, DOUT), lambda b: (b, 0, 0)),
            pl.BlockSpec((1, 1, NKEEP), lambda b: (b, 0, 0)),
            pl.BlockSpec((1, 1, N), lambda b: (b, 0, 0)),
        ],
        out_shape=[
            jax.ShapeDtypeStruct((B, N, DOUT), f32),
            jax.ShapeDtypeStruct((B, 1, NKEEP), i32),
            jax.ShapeDtypeStruct((B, 1, N), i32),
        ],
    )(A, X, Wr, Wm, bm)


_TOT = B * NKEEP
_NW = 32
_RPW = _TOT // _NW
_ACH = 32


def _sc_body(a2, xo2, gid, rnkf, aout, xout,
             gid_v, inv_v, xbuf, abuf, obuf, sem):
    wid = lax.axis_index("s") * 2 + lax.axis_index("c")
    base = wid * _RPW
    b = base // NKEEP
    pltpu.sync_copy(gid.at[pl.ds(base, _RPW)], gid_v)
    pltpu.sync_copy(rnkf.at[pl.ds(b * N, N)], inv_v)

    pltpu.async_copy(xo2.at[gid_v], xbuf, sem).wait()
    pltpu.sync_copy(xbuf, xout.at[pl.ds(base, _RPW)])

    for ci in range(_RPW // _ACH):
        pltpu.async_copy(a2.at[gid_v.at[pl.ds(ci * _ACH, _ACH)]], abuf,
                         sem).wait()

        lane = lax.broadcasted_iota(i32, (16,), 0)

        def row_body(r, carry):
            rbase = r * NKEEP
            for k in range(N // 16):
                vals = abuf[r, pl.ds(k * 16, 16)]
                invv = inv_v[pl.ds(k * 16, 16)]
                pos = jnp.where(invv < NKEEP, invv + rbase,
                                _ACH * NKEEP + lane)
                plsc.store_scatter(obuf, [pos], vals)
            return carry

        lax.fori_loop(0, _ACH, row_body, 0)
        pltpu.sync_copy(
            obuf.at[pl.ds(0, _ACH * NKEEP)],
            aout.at[pl.ds((base + ci * _ACH) * NKEEP, _ACH * NKEEP)])


def _sc_stage(A2, XO2, gidf, rnkf):
    mesh = plsc.VectorSubcoreMesh(core_axis_name="c", subcore_axis_name="s",
                                  num_cores=2)
    kern = functools.partial(
        pl.kernel,
        mesh=mesh,
        compiler_params=pltpu.CompilerParams(needs_layout_passes=False),
        out_type=[
            jax.ShapeDtypeStruct((_TOT * NKEEP,), f32),
            jax.ShapeDtypeStruct((_TOT, DOUT), f32),
        ],
        scratch_types=[
            pltpu.VMEM((_RPW,), i32),
            pltpu.VMEM((N,), i32),
            pltpu.VMEM((_RPW, DOUT), f32),
            pltpu.VMEM((_ACH, N), f32),
            pltpu.VMEM((_ACH * NKEEP + 16,), f32),
            pltpu.SemaphoreType.DMA,
        ],
    )(_sc_body)
    return kern(A2, XO2, gidf, rnkf)


def kernel(A, X, W_rank, b_rank, W_model, b_model):
    xo, gid, rnk = _tc_stage(A, X, W_rank, W_model, b_model)
    aof, xof = _sc_stage(
        A.reshape(B * N, N),
        xo.reshape(B * N, DOUT),
        gid.reshape(_TOT),
        rnk.reshape(B * N),
    )
    A_out = jnp.transpose(aof.reshape(B, NKEEP, NKEEP), (0, 2, 1))
    return (A_out, xof.reshape(B, NKEEP, DOUT))


# speedup vs baseline: 1.0765x; 1.0765x over previous
import functools

import jax
import jax.numpy as jnp
from jax import lax
from jax.experimental import pallas as pl
from jax.experimental.pallas import tpu as pltpu
from jax.experimental.pallas import tpu_sc as plsc

B, N, DIN, DOUT, NKEEP = 8, 1024, 256, 256, 512
bf16, f32, i32 = jnp.bfloat16, jnp.float32, jnp.int32


def _tc_body(a_ref, x_ref, wr_ref, wm_ref, bm_ref,
             xo_ref, gid_ref, rnk_ref):
    b = pl.program_id(0)
    a = a_ref[0].astype(bf16)
    x = x_ref[0].astype(bf16)
    mt = lax.dot_general(x, a, (((0,), (1,)), ((), ())),
                         preferred_element_type=f32)
    m_bf = mt.T.astype(bf16)
    wr_bf = wr_ref[...].astype(bf16)
    t_col = jnp.dot(m_bf, wr_bf.T, preferred_element_type=f32)
    t_row = lax.dot_general(wr_bf.T, m_bf, (((0,), (1,)), ((), ())),
                            preferred_element_type=f32)

    wm_bf = wm_ref[...].astype(bf16)
    xo = jnp.dot(m_bf, wm_bf.T, preferred_element_type=f32) + bm_ref[...]
    xo_ref[0] = jnp.where(xo >= 0, xo, 0.01 * xo)

    ii = lax.broadcasted_iota(i32, (N, N), 0)
    jj = lax.broadcasted_iota(i32, (N, N), 1)
    beats = (t_row > t_col) | ((t_row == t_col) & (jj < ii))
    rank_col = jnp.sum(jnp.where(beats, 1.0, 0.0), axis=1,
                       keepdims=True)
    rnk_ref[0] = rank_col.astype(i32)

    pmat = lax.broadcasted_iota(i32, (N, NKEEP), 1).astype(f32)
    emat = lax.broadcasted_iota(i32, (N, NKEEP), 0).astype(f32)
    onehot = rank_col == pmat
    idx_row = jnp.sum(jnp.where(onehot, emat, 0.0), axis=0,
                      keepdims=True)
    gid_ref[0] = idx_row.astype(i32) + b * N


def _tc_stage(A, X, Wr, Wm, bm):
    return pl.pallas_call(
        _tc_body,
        grid=(B,),
        in_specs=[
            pl.BlockSpec((1, N, N), lambda b: (b, 0, 0)),
            pl.BlockSpec((1, N, DIN), lambda b: (b, 0, 0)),
            pl.BlockSpec((1, DIN), lambda b: (0, 0)),
            pl.BlockSpec((DOUT, DIN), lambda b: (0, 0)),
            pl.BlockSpec((DOUT,), lambda b: (0,)),
        ],
        out_specs=[
            pl.BlockSpec((1, N, DOUT), lambda b: (b, 0, 0)),
            pl.BlockSpec((1, 1, NKEEP), lambda b: (b, 0, 0)),
            pl.BlockSpec((1, N, 1), lambda b: (b, 0, 0)),
        ],
        out_shape=[
            jax.ShapeDtypeStruct((B, N, DOUT), f32),
            jax.ShapeDtypeStruct((B, 1, NKEEP), i32),
            jax.ShapeDtypeStruct((B, N, 1), i32),
        ],
    )(A, X, Wr, Wm, bm)


_TOT = B * NKEEP
_NW = 32
_RPW = _TOT // _NW
_ACH = 32
_NCH = _RPW // _ACH


def _sc_body(a2, xo2, gid, rnkf, aout, xout,
             gid_v, inv_v, pos_v, xbuf, abuf0, abuf1, obuf, sem_x, sem_a):
    wid = lax.axis_index("s") * 2 + lax.axis_index("c")
    base = wid * _RPW
    b = base // NKEEP
    pltpu.sync_copy(gid.at[pl.ds(base, _RPW)], gid_v)
    pltpu.sync_copy(rnkf.at[pl.ds(b * N, N)], inv_v)

    copy_x = pltpu.async_copy(xo2.at[gid_v], xbuf, sem_x)

    lane = lax.broadcasted_iota(i32, (16,), 0)
    for k in range(N // 16):
        invv = inv_v[pl.ds(k * 16, 16)]
        pos_v[pl.ds(k * 16, 16)] = jnp.where(invv < NKEEP, invv,
                                             NKEEP + lane)

    bufs = (abuf0, abuf1)
    copies = [None] * _NCH
    copies[0] = pltpu.async_copy(a2.at[gid_v.at[pl.ds(0, _ACH)]], abuf0,
                                 sem_a)
    for ci in range(_NCH):
        if ci + 1 < _NCH:
            copies[ci + 1] = pltpu.async_copy(
                a2.at[gid_v.at[pl.ds((ci + 1) * _ACH, _ACH)]],
                bufs[(ci + 1) % 2], sem_a)
        copies[ci].wait()
        abuf = bufs[ci % 2]

        def row_body(r, carry):
            rbase = r * NKEEP
            for k in range(N // 16):
                vals = abuf[r, pl.ds(k * 16, 16)]
                pos = pos_v[pl.ds(k * 16, 16)] + rbase
                plsc.store_scatter(obuf, [pos], vals)
            return carry

        lax.fori_loop(0, _ACH, row_body, 0)
        pltpu.sync_copy(
            obuf.at[pl.ds(0, _ACH * NKEEP)],
            aout.at[pl.ds((base + ci * _ACH) * NKEEP, _ACH * NKEEP)])

    copy_x.wait()
    pltpu.sync_copy(xbuf, xout.at[pl.ds(base, _RPW)])


def _sc_stage(A2, XO2, gidf, rnkf):
    mesh = plsc.VectorSubcoreMesh(core_axis_name="c", subcore_axis_name="s",
                                  num_cores=2)
    kern = functools.partial(
        pl.kernel,
        mesh=mesh,
        compiler_params=pltpu.CompilerParams(needs_layout_passes=False),
        out_type=[
            jax.ShapeDtypeStruct((_TOT * NKEEP,), f32),
            jax.ShapeDtypeStruct((_TOT, DOUT), f32),
        ],
        scratch_types=[
            pltpu.VMEM((_RPW,), i32),
            pltpu.VMEM((N,), i32),
            pltpu.VMEM((N,), i32),
            pltpu.VMEM((_RPW, DOUT), f32),
            pltpu.VMEM((_ACH, N), f32),
            pltpu.VMEM((_ACH, N), f32),
            pltpu.VMEM((_ACH * NKEEP + 16,), f32),
            pltpu.SemaphoreType.DMA,
            pltpu.SemaphoreType.DMA,
        ],
    )(_sc_body)
    return kern(A2, XO2, gidf, rnkf)


def kernel(A, X, W_rank, b_rank, W_model, b_model):
    xo, gid, rnk = _tc_stage(A, X, W_rank, W_model, b_model)
    aof, xof = _sc_stage(
        A.reshape(B * N, N),
        xo.reshape(B * N, DOUT),
        gid.reshape(_TOT),
        rnk.reshape(B * N),
    )
    A_out = jnp.transpose(aof.reshape(B, NKEEP, NKEEP), (0, 2, 1))
    return (A_out, xof.reshape(B, NKEEP, DOUT))


# speedup vs baseline: 1.1346x; 1.0540x over previous
import functools

import jax
import jax.numpy as jnp
from jax import lax
from jax.experimental import pallas as pl
from jax.experimental.pallas import tpu as pltpu
from jax.experimental.pallas import tpu_sc as plsc

B, N, DIN, DOUT, NKEEP = 8, 1024, 256, 256, 512
bf16, f32, i32 = jnp.bfloat16, jnp.float32, jnp.int32


def _tc_body(a_ref, x_ref, wr_ref, wm_ref, bm_ref,
             xo_ref, gid_ref):
    b = pl.program_id(0)
    a = a_ref[0].astype(bf16)
    x = x_ref[0].astype(bf16)
    mt = lax.dot_general(x, a, (((0,), (1,)), ((), ())),
                         preferred_element_type=f32)
    m_bf = mt.T.astype(bf16)
    wr_bf = wr_ref[...].astype(bf16)
    t_col = jnp.dot(m_bf, wr_bf.T, preferred_element_type=f32)
    t_row = lax.dot_general(wr_bf.T, m_bf, (((0,), (1,)), ((), ())),
                            preferred_element_type=f32)

    wm_bf = wm_ref[...].astype(bf16)
    xo = jnp.dot(m_bf, wm_bf.T, preferred_element_type=f32) + bm_ref[...]
    xo_ref[0] = jnp.where(xo >= 0, xo, 0.01 * xo)

    ii = lax.broadcasted_iota(i32, (N, N), 0)
    jj = lax.broadcasted_iota(i32, (N, N), 1)
    beats = (t_row > t_col) | ((t_row == t_col) & (jj < ii))
    rank_col = jnp.sum(jnp.where(beats, 1.0, 0.0), axis=1,
                       keepdims=True)

    pmat = lax.broadcasted_iota(i32, (N, NKEEP), 1).astype(f32)
    emat = lax.broadcasted_iota(i32, (N, NKEEP), 0).astype(f32)
    onehot = rank_col == pmat
    idx_row = jnp.sum(jnp.where(onehot, emat, 0.0), axis=0,
                      keepdims=True)
    gid_ref[0] = idx_row.astype(i32) + b * N


def _tc_stage(A, X, Wr, Wm, bm):
    return pl.pallas_call(
        _tc_body,
        grid=(B,),
        in_specs=[
            pl.BlockSpec((1, N, N), lambda b: (b, 0, 0)),
            pl.BlockSpec((1, N, DIN), lambda b: (b, 0, 0)),
            pl.BlockSpec((1, DIN), lambda b: (0, 0)),
            pl.BlockSpec((DOUT, DIN), lambda b: (0, 0)),
            pl.BlockSpec((DOUT,), lambda b: (0,)),
        ],
        out_specs=[
            pl.BlockSpec((1, N, DOUT), lambda b: (b, 0, 0)),
            pl.BlockSpec((1, 1, NKEEP), lambda b: (b, 0, 0)),
        ],
        out_shape=[
            jax.ShapeDtypeStruct((B, N, DOUT), f32),
            jax.ShapeDtypeStruct((B, 1, NKEEP), i32),
        ],
    )(A, X, Wr, Wm, bm)


_TOT = B * NKEEP
_NW = 32
_RPW = _TOT // _NW
_ACH = 32
_NCH = _RPW // _ACH


def _sc_body(a2, xo2, gid, aout, xout,
             gid_v, pos_v, xbuf, abuf0, abuf1, obuf, sem_x, sem_a):
    wid = lax.axis_index("s") * 2 + lax.axis_index("c")
    base = wid * _RPW
    b = base // NKEEP
    pltpu.sync_copy(gid.at[pl.ds(base, _RPW)], gid_v)
    pltpu.sync_copy(gid.at[pl.ds(b * NKEEP, NKEEP)], pos_v)

    copy_x = pltpu.async_copy(xo2.at[gid_v], xbuf, sem_x)

    for k in range(NKEEP // 16):
        pos_v[pl.ds(k * 16, 16)] = pos_v[pl.ds(k * 16, 16)] - b * N

    bufs = (abuf0, abuf1)
    copies = [None] * _NCH
    copies[0] = pltpu.async_copy(a2.at[gid_v.at[pl.ds(0, _ACH)]], abuf0,
                                 sem_a)
    for ci in range(_NCH):
        if ci + 1 < _NCH:
            copies[ci + 1] = pltpu.async_copy(
                a2.at[gid_v.at[pl.ds((ci + 1) * _ACH, _ACH)]],
                bufs[(ci + 1) % 2], sem_a)
        copies[ci].wait()
        abuf = bufs[ci % 2]

        def row_body(r, carry):
            rbase = r * NKEEP
            row_idx = jnp.full((16,), r, i32)
            for k in range(NKEEP // 16):
                colv = pos_v[pl.ds(k * 16, 16)]
                obuf[pl.ds(rbase + k * 16, 16)] = plsc.load_gather(
                    abuf, [row_idx, colv])
            return carry

        lax.fori_loop(0, _ACH, row_body, 0)
        pltpu.sync_copy(
            obuf.at[pl.ds(0, _ACH * NKEEP)],
            aout.at[pl.ds((base + ci * _ACH) * NKEEP, _ACH * NKEEP)])

    copy_x.wait()
    pltpu.sync_copy(xbuf, xout.at[pl.ds(base, _RPW)])


def _sc_stage(A2, XO2, gidf):
    mesh = plsc.VectorSubcoreMesh(core_axis_name="c", subcore_axis_name="s",
                                  num_cores=2)
    kern = functools.partial(
        pl.kernel,
        mesh=mesh,
        compiler_params=pltpu.CompilerParams(needs_layout_passes=False),
        out_type=[
            jax.ShapeDtypeStruct((_TOT * NKEEP,), f32),
            jax.ShapeDtypeStruct((_TOT, DOUT), f32),
        ],
        scratch_types=[
            pltpu.VMEM((_RPW,), i32),
            pltpu.VMEM((NKEEP,), i32),
            pltpu.VMEM((_RPW, DOUT), f32),
            pltpu.VMEM((_ACH, N), f32),
            pltpu.VMEM((_ACH, N), f32),
            pltpu.VMEM((_ACH * NKEEP,), f32),
            pltpu.SemaphoreType.DMA,
            pltpu.SemaphoreType.DMA,
        ],
    )(_sc_body)
    return kern(A2, XO2, gidf)


def kernel(A, X, W_rank, b_rank, W_model, b_model):
    xo, gid = _tc_stage(A, X, W_rank, W_model, b_model)
    aof, xof = _sc_stage(
        A.reshape(B * N, N),
        xo.reshape(B * N, DOUT),
        gid.reshape(_TOT),
    )
    A_out = jnp.transpose(aof.reshape(B, NKEEP, NKEEP), (0, 2, 1))
    return (A_out, xof.reshape(B, NKEEP, DOUT))
